# Initial kernel scaffold; baseline (speedup 1.0000x reference)
import functools

import jax
import jax.numpy as jnp
from jax import lax
from jax.experimental import pallas as pl
from jax.experimental.pallas import tpu as pltpu
from jax.experimental.pallas import tpu_sc as plsc

N, E, D = 10000, 320000, 128
NC, NS = 2, 16
EPC = E // NC
EPW = EPC // NS
CH = 128
NFULL = EPW // CH
TAIL = EPW - NFULL * CH
RPW = N // NS
ZR = 125


def _linear(h, w, b):
    def body(h_ref, w_ref, b_ref, o_ref):
        o_ref[...] = lax.dot_general(
            h_ref[...], w_ref[...], (((1,), (1,)), ((), ())),
            preferred_element_type=jnp.float32) + b_ref[...]

    return pl.pallas_call(
        body,
        out_shape=jax.ShapeDtypeStruct((N, D), jnp.float32),
    )(h, w, b.reshape(1, D))


def _sc_aggregate(Bh, src, dst):
    mesh = plsc.VectorSubcoreMesh(core_axis_name="c", subcore_axis_name="s")

    @functools.partial(
        pl.kernel,
        out_type=jax.ShapeDtypeStruct((NC, N, D), jnp.float32),
        mesh=mesh,
        scratch_types=[
            pltpu.VMEM((CH,), jnp.int32),
            pltpu.VMEM((CH,), jnp.int32),
            pltpu.VMEM((CH, D), jnp.float32),
            pltpu.VMEM((TAIL,), jnp.int32),
            pltpu.VMEM((TAIL,), jnp.int32),
            pltpu.VMEM((TAIL, D), jnp.float32),
            pltpu.VMEM((ZR, D), jnp.float32),
            pltpu.VMEM_SHARED((N, D), jnp.float32),
            pltpu.SemaphoreType.DMA,
        ],
    )
    def k(bh_hbm, src_hbm, dst_hbm, out_hbm,
          sidx, didx, rows, tsidx, tdidx, trows, zbuf, acc, sem):
        cid = lax.axis_index("c")
        sid = lax.axis_index("s")

        @pl.loop(0, ZR)
        def _(r):
            @pl.loop(0, D, step=16)
            def _(c):
                zbuf[r, pl.ds(c, 16)] = jnp.zeros((16,), jnp.float32)

        @pl.loop(0, RPW, step=ZR)
        def _(r):
            pltpu.sync_copy(zbuf, acc.at[pl.ds(sid * RPW + r, ZR)])

        plsc.subcore_barrier()

        ebase = cid * EPC + sid * EPW

        @pl.loop(0, NFULL * CH, step=CH)
        def _(i):
            off = ebase + i
            pltpu.sync_copy(src_hbm.at[pl.ds(off, CH)], sidx)
            pltpu.sync_copy(dst_hbm.at[pl.ds(off, CH)], didx)
            pltpu.async_copy(bh_hbm.at[sidx], rows, sem).wait()
            pltpu.sync_copy(rows, acc.at[didx], add=True)

        toff = ebase + NFULL * CH
        pltpu.sync_copy(src_hbm.at[pl.ds(toff, TAIL)], tsidx)
        pltpu.sync_copy(dst_hbm.at[pl.ds(toff, TAIL)], tdidx)
        pltpu.async_copy(bh_hbm.at[tsidx], trows, sem).wait()
        pltpu.sync_copy(trows, acc.at[tdidx], add=True)

        plsc.subcore_barrier()

        pltpu.sync_copy(acc.at[pl.ds(sid * RPW, RPW)],
                        out_hbm.at[cid, pl.ds(sid * RPW, RPW)])

    return k(Bh, src, dst)


def _epilogue(h, A_w, A_b, partials, gamma, beta):
    def body(h_ref, aw_ref, ab_ref, p_ref, g_ref, b_ref, o_ref):
        hv = h_ref[...]
        ah = lax.dot_general(
            hv, aw_ref[...], (((1,), (1,)), ((), ())),
            preferred_element_type=jnp.float32)
        hn = ah + ab_ref[...] + p_ref[0] + p_ref[1]
        mean = jnp.sum(hn, axis=0, keepdims=True) / N
        sq = jnp.sum(hn * hn, axis=0, keepdims=True) / N
        var = sq - mean * mean
        inv = lax.rsqrt(var + 1e-5) * g_ref[...]
        bn = (hn - mean) * inv + b_ref[...]
        o_ref[...] = hv + jnp.maximum(bn, 0.0)

    return pl.pallas_call(
        body,
        out_shape=jax.ShapeDtypeStruct((N, D), jnp.float32),
    )(h, A_w, A_b.reshape(1, D), partials, gamma.reshape(1, D),
      beta.reshape(1, D))


def kernel(h, edge_index, e, A_w, A_b, B_w, B_b, gamma, beta):
    Bh = _linear(h, B_w, B_b)
    partials = _sc_aggregate(Bh, edge_index[0], edge_index[1])
    hn = _epilogue(h, A_w, A_b, partials, gamma, beta)
    return (hn, e)


# speedup vs baseline: 6.0688x; 6.0688x over previous
import functools

import jax
import jax.numpy as jnp
from jax import lax
from jax.experimental import pallas as pl
from jax.experimental.pallas import tpu as pltpu
from jax.experimental.pallas import tpu_sc as plsc

N, E, D = 10000, 320000, 128
NC, NS = 2, 16
EPC = E // NC
EPW = EPC // NS
CH = 128
NFULL = EPW // CH
TAIL = EPW - NFULL * CH
RPW = 624
RTAIL = N - NS * RPW
WB = 208


def _linear(h, w, b):
    def body(h_ref, w_ref, b_ref, o_ref):
        o_ref[...] = lax.dot_general(
            h_ref[...], w_ref[...], (((1,), (1,)), ((), ())),
            preferred_element_type=jnp.float32) + b_ref[...]

    return pl.pallas_call(
        body,
        out_shape=jax.ShapeDtypeStruct((N, D), jnp.float32),
    )(h, w, b.reshape(1, D))


def _sc_aggregate(Bh, src, dst):
    mesh = plsc.VectorSubcoreMesh(core_axis_name="c", subcore_axis_name="s")

    @functools.partial(
        pl.kernel,
        out_type=jax.ShapeDtypeStruct((NC, N, D), jnp.float32),
        mesh=mesh,
        scratch_types=[
            pltpu.VMEM((CH,), jnp.int32),
            pltpu.VMEM((CH,), jnp.int32),
            pltpu.VMEM((CH, D), jnp.float32),
            pltpu.VMEM((TAIL,), jnp.int32),
            pltpu.VMEM((TAIL,), jnp.int32),
            pltpu.VMEM((TAIL, D), jnp.float32),
            pltpu.VMEM((WB, D), jnp.float32),
            pltpu.VMEM_SHARED((N, D), jnp.float32),
            pltpu.SemaphoreType.DMA,
        ],
    )
    def k(bh_hbm, src_hbm, dst_hbm, out_hbm,
          sidx, didx, rows, tsidx, tdidx, trows, zbuf, acc, sem):
        cid = lax.axis_index("c")
        sid = lax.axis_index("s")

        @pl.loop(0, WB)
        def _(r):
            @pl.loop(0, D, step=16)
            def _(c):
                zbuf[r, pl.ds(c, 16)] = jnp.zeros((16,), jnp.float32)

        @pl.loop(0, RPW, step=WB)
        def _(r):
            pltpu.sync_copy(zbuf, acc.at[pl.ds(sid * RPW + r, WB)])

        @pl.when(sid == 0)
        def _():
            pltpu.sync_copy(zbuf.at[pl.ds(0, RTAIL)],
                            acc.at[pl.ds(NS * RPW, RTAIL)])

        plsc.subcore_barrier()

        ebase = cid * EPC + sid * EPW

        @pl.loop(0, NFULL * CH, step=CH)
        def _(i):
            off = ebase + i
            pltpu.sync_copy(src_hbm.at[pl.ds(off, CH)], sidx)
            pltpu.sync_copy(dst_hbm.at[pl.ds(off, CH)], didx)
            pltpu.async_copy(bh_hbm.at[sidx], rows, sem).wait()
            pltpu.sync_copy(rows, acc.at[didx], add=True)

        toff = ebase + NFULL * CH
        pltpu.sync_copy(src_hbm.at[pl.ds(toff, TAIL)], tsidx)
        pltpu.sync_copy(dst_hbm.at[pl.ds(toff, TAIL)], tdidx)
        pltpu.async_copy(bh_hbm.at[tsidx], trows, sem).wait()
        pltpu.sync_copy(trows, acc.at[tdidx], add=True)

        plsc.subcore_barrier()

        @pl.loop(0, RPW, step=WB)
        def _(r):
            pltpu.sync_copy(acc.at[pl.ds(sid * RPW + r, WB)],
                            out_hbm.at[cid, pl.ds(sid * RPW + r, WB)])

        @pl.when(sid == 0)
        def _():
            pltpu.sync_copy(acc.at[pl.ds(NS * RPW, RTAIL)],
                            out_hbm.at[cid, pl.ds(NS * RPW, RTAIL)])

    return k(Bh, src, dst)


def _epilogue(h, A_w, A_b, partials, gamma, beta):
    def body(h_ref, aw_ref, ab_ref, p_ref, g_ref, b_ref, o_ref):
        hv = h_ref[...]
        ah = lax.dot_general(
            hv, aw_ref[...], (((1,), (1,)), ((), ())),
            preferred_element_type=jnp.float32)
        hn = ah + ab_ref[...] + p_ref[0] + p_ref[1]
        mean = jnp.sum(hn, axis=0, keepdims=True) / N
        sq = jnp.sum(hn * hn, axis=0, keepdims=True) / N
        var = sq - mean * mean
        inv = lax.rsqrt(var + 1e-5) * g_ref[...]
        bn = (hn - mean) * inv + b_ref[...]
        o_ref[...] = hv + jnp.maximum(bn, 0.0)

    return pl.pallas_call(
        body,
        out_shape=jax.ShapeDtypeStruct((N, D), jnp.float32),
    )(h, A_w, A_b.reshape(1, D), partials, gamma.reshape(1, D),
      beta.reshape(1, D))


def kernel(h, edge_index, e, A_w, A_b, B_w, B_b, gamma, beta):
    Bh = _linear(h, B_w, B_b)
    partials = _sc_aggregate(Bh, edge_index[0], edge_index[1])
    hn = _epilogue(h, A_w, A_b, partials, gamma, beta)
    return (hn, e)


# speedup vs baseline: 6.9069x; 1.1381x over previous
import functools

import jax
import jax.numpy as jnp
from jax import lax
from jax.experimental import pallas as pl
from jax.experimental.pallas import tpu as pltpu
from jax.experimental.pallas import tpu_sc as plsc

N, E, D = 10000, 320000, 128
NC, NS = 2, 16
EPC = E // NC
EPW = EPC // NS
CH = 64
NPAIR = EPW // (2 * CH)
TAIL = EPW - NPAIR * 2 * CH
RPW = 624
RTAIL = N - NS * RPW
WB = 208


def _linear(h, w, b):
    def body(h_ref, w_ref, b_ref, o_ref):
        o_ref[...] = lax.dot_general(
            h_ref[...], w_ref[...], (((1,), (1,)), ((), ())),
            preferred_element_type=jnp.float32) + b_ref[...]

    return pl.pallas_call(
        body,
        out_shape=jax.ShapeDtypeStruct((N, D), jnp.float32),
    )(h, w, b.reshape(1, D))


def _sc_aggregate(Bh, src, dst):
    mesh = plsc.VectorSubcoreMesh(core_axis_name="c", subcore_axis_name="s")

    @functools.partial(
        pl.kernel,
        out_type=jax.ShapeDtypeStruct((NC, N, D), jnp.float32),
        mesh=mesh,
        scratch_types=[
            pltpu.VMEM((CH,), jnp.int32),
            pltpu.VMEM((CH,), jnp.int32),
            pltpu.VMEM((CH,), jnp.int32),
            pltpu.VMEM((CH,), jnp.int32),
            pltpu.VMEM((CH, D), jnp.float32),
            pltpu.VMEM((CH, D), jnp.float32),
            pltpu.VMEM((TAIL,), jnp.int32),
            pltpu.VMEM((TAIL,), jnp.int32),
            pltpu.VMEM((TAIL, D), jnp.float32),
            pltpu.VMEM((WB, D), jnp.float32),
            pltpu.VMEM_SHARED((N, D), jnp.float32),
            pltpu.SemaphoreType.DMA,
            pltpu.SemaphoreType.DMA,
        ],
    )
    def k(bh_hbm, src_hbm, dst_hbm, out_hbm,
          sidx0, sidx1, didx0, didx1, rows0, rows1, tsidx, tdidx, trows,
          zbuf, acc, sem0, sem1):
        cid = lax.axis_index("c")
        sid = lax.axis_index("s")

        @pl.loop(0, WB)
        def _(r):
            @pl.loop(0, D, step=16)
            def _(c):
                zbuf[r, pl.ds(c, 16)] = jnp.zeros((16,), jnp.float32)

        @pl.loop(0, RPW, step=WB)
        def _(r):
            pltpu.sync_copy(zbuf, acc.at[pl.ds(sid * RPW + r, WB)])

        @pl.when(sid == 0)
        def _():
            pltpu.sync_copy(zbuf.at[pl.ds(0, RTAIL)],
                            acc.at[pl.ds(NS * RPW, RTAIL)])

        plsc.subcore_barrier()

        ebase = cid * EPC + sid * EPW

        @pl.loop(0, NPAIR * 2 * CH, step=2 * CH)
        def _(i):
            off = ebase + i
            pltpu.sync_copy(src_hbm.at[pl.ds(off, CH)], sidx0)
            c0 = pltpu.async_copy(bh_hbm.at[sidx0], rows0, sem0)
            pltpu.sync_copy(src_hbm.at[pl.ds(off + CH, CH)], sidx1)
            c1 = pltpu.async_copy(bh_hbm.at[sidx1], rows1, sem1)
            pltpu.sync_copy(dst_hbm.at[pl.ds(off, CH)], didx0)
            pltpu.sync_copy(dst_hbm.at[pl.ds(off + CH, CH)], didx1)
            c0.wait()
            pltpu.sync_copy(rows0, acc.at[didx0], add=True)
            c1.wait()
            pltpu.sync_copy(rows1, acc.at[didx1], add=True)

        toff = ebase + NPAIR * 2 * CH
        pltpu.sync_copy(src_hbm.at[pl.ds(toff, TAIL)], tsidx)
        pltpu.sync_copy(dst_hbm.at[pl.ds(toff, TAIL)], tdidx)
        pltpu.async_copy(bh_hbm.at[tsidx], trows, sem0).wait()
        pltpu.sync_copy(trows, acc.at[tdidx], add=True)

        plsc.subcore_barrier()

        @pl.loop(0, RPW, step=WB)
        def _(r):
            pltpu.sync_copy(acc.at[pl.ds(sid * RPW + r, WB)],
                            out_hbm.at[cid, pl.ds(sid * RPW + r, WB)])

        @pl.when(sid == 0)
        def _():
            pltpu.sync_copy(acc.at[pl.ds(NS * RPW, RTAIL)],
                            out_hbm.at[cid, pl.ds(NS * RPW, RTAIL)])

    return k(Bh, src, dst)


def _epilogue(h, A_w, A_b, partials, gamma, beta):
    def body(h_ref, aw_ref, ab_ref, p_ref, g_ref, b_ref, o_ref):
        hv = h_ref[...]
        ah = lax.dot_general(
            hv, aw_ref[...], (((1,), (1,)), ((), ())),
            preferred_element_type=jnp.float32)
        hn = ah + ab_ref[...] + p_ref[0] + p_ref[1]
        mean = jnp.sum(hn, axis=0, keepdims=True) / N
        sq = jnp.sum(hn * hn, axis=0, keepdims=True) / N
        var = sq - mean * mean
        inv = lax.rsqrt(var + 1e-5) * g_ref[...]
        bn = (hn - mean) * inv + b_ref[...]
        o_ref[...] = hv + jnp.maximum(bn, 0.0)

    return pl.pallas_call(
        body,
        out_shape=jax.ShapeDtypeStruct((N, D), jnp.float32),
    )(h, A_w, A_b.reshape(1, D), partials, gamma.reshape(1, D),
      beta.reshape(1, D))


def kernel(h, edge_index, e, A_w, A_b, B_w, B_b, gamma, beta):
    Bh = _linear(h, B_w, B_b)
    partials = _sc_aggregate(Bh, edge_index[0], edge_index[1])
    hn = _epilogue(h, A_w, A_b, partials, gamma, beta)
    return (hn, e)
